# Initial kernel scaffold; baseline (speedup 1.0000x reference)
import jax
import jax.numpy as jnp
from jax.experimental import pallas as pl


def kernel(user_ids, item_ids, user_feats, item_feats, W_user, W_item, W_ufeat, W_ifeat, user_bias, item_bias, user_feat_bias, item_feat_bias, offset, A, Bmat):
    raise NotImplementedError("write your pallas kernel here")


# speedup vs baseline: 1.6773x; 1.6773x over previous
import functools

import jax
import jax.numpy as jnp
from jax import lax
from jax.experimental import pallas as pl
from jax.experimental.pallas import tpu as pltpu
from jax.experimental.pallas import tpu_sc as plsc

B = 16384
F = 32
R = 16
NC = 2
NS = 16
NW = NC * NS
BPW = B // NW

BB = 2048
NB = B // BB


def _sc_gather_body(uid_hbm, iid_hbm, wu_hbm, wi_hbm, a_hbm, ub_hbm, ib_hbm,
                    ue_out, ie_out, a_out, ub_out, ib_out,
                    uidx_v, iidx_v, ue_v, wiv_v, a_v, ub_v, ib_v, sem):
    wid = lax.axis_index("s") * NC + lax.axis_index("c")
    base = wid * BPW
    pltpu.sync_copy(uid_hbm.at[pl.ds(base, BPW)], uidx_v)
    pltpu.sync_copy(iid_hbm.at[pl.ds(base, BPW)], iidx_v)
    c1 = pltpu.async_copy(wu_hbm.at[uidx_v], ue_v, sem)
    c2 = pltpu.async_copy(wi_hbm.at[iidx_v], wiv_v, sem)
    c3 = pltpu.async_copy(a_hbm.at[iidx_v], a_v, sem)
    c4 = pltpu.async_copy(ub_hbm.at[uidx_v], ub_v, sem)
    c5 = pltpu.async_copy(ib_hbm.at[iidx_v], ib_v, sem)
    c1.wait(); c2.wait(); c3.wait(); c4.wait(); c5.wait()
    pltpu.sync_copy(ue_v, ue_out.at[pl.ds(base, BPW)])
    pltpu.sync_copy(wiv_v, ie_out.at[pl.ds(base, BPW)])
    pltpu.sync_copy(a_v, a_out.at[pl.ds(base, BPW)])
    pltpu.sync_copy(ub_v, ub_out.at[pl.ds(base, BPW)])
    pltpu.sync_copy(ib_v, ib_out.at[pl.ds(base, BPW)])


@functools.cache
def _make_sc_gather():
    return pl.kernel(
        _sc_gather_body,
        out_type=(
            jax.ShapeDtypeStruct((B, F), jnp.float32),
            jax.ShapeDtypeStruct((B, F), jnp.float32),
            jax.ShapeDtypeStruct((B, R), jnp.float32),
            jax.ShapeDtypeStruct((B,), jnp.float32),
            jax.ShapeDtypeStruct((B,), jnp.float32),
        ),
        mesh=plsc.VectorSubcoreMesh(core_axis_name="c", subcore_axis_name="s",
                                    num_cores=NC, num_subcores=NS),
        compiler_params=pltpu.CompilerParams(use_tc_tiling_on_sc=False),
        scratch_types=[
            pltpu.VMEM((BPW,), jnp.int32),
            pltpu.VMEM((BPW,), jnp.int32),
            pltpu.VMEM((BPW, F), jnp.float32),
            pltpu.VMEM((BPW, F), jnp.float32),
            pltpu.VMEM((BPW, R), jnp.float32),
            pltpu.VMEM((BPW,), jnp.float32),
            pltpu.VMEM((BPW,), jnp.float32),
            pltpu.SemaphoreType.DMA,
        ],
    )


def _tc_body(uf_ref, itf_ref, ue_ref, wiv_ref, a_ref, ub_ref, ib_ref,
             wuf_ref, wif_ref, bmat_ref, ufb_ref, ifb_ref, off_ref, out_ref):
    uf = uf_ref[...]
    itf = itf_ref[...]
    ue = ue_ref[...]
    wiv = wiv_ref[...]
    a = a_ref[...]
    wuf = wuf_ref[...]
    wif = wif_ref[...]
    bmat = bmat_ref[...]

    ie = wiv + jnp.dot(a, bmat, preferred_element_type=jnp.float32)
    dsum = (jnp.dot(uf, wuf, preferred_element_type=jnp.float32)
            + jnp.dot(itf, wif, preferred_element_type=jnp.float32))
    s = ue + ie + dsum

    wuf2 = jnp.sum(wuf * wuf, axis=1)
    wif2 = jnp.sum(wif * wif, axis=1)
    sq = (jnp.sum(ue * ue, axis=1, keepdims=True)
          + jnp.sum(ie * ie, axis=1, keepdims=True)
          + jnp.sum(uf * uf * wuf2[None, :], axis=1, keepdims=True)
          + jnp.sum(itf * itf * wif2[None, :], axis=1, keepdims=True))
    quad = jnp.sum(s * s, axis=1, keepdims=True) - sq

    fb = (jnp.sum(uf * ufb_ref[...], axis=1, keepdims=True)
          + jnp.sum(itf * ifb_ref[...], axis=1, keepdims=True))
    out_ref[...] = (0.5 * quad + ub_ref[...] + ib_ref[...] + fb
                    + off_ref[0, 0])


def kernel(user_ids, item_ids, user_feats, item_feats, W_user, W_item,
           W_ufeat, W_ifeat, user_bias, item_bias, user_feat_bias,
           item_feat_bias, offset, A, Bmat):
    uid = user_ids.astype(jnp.int32)
    iid = item_ids.astype(jnp.int32)
    ue, wiv, a, ub, ib = _make_sc_gather()(uid, iid, W_user, W_item, A,
                                           user_bias, item_bias)

    nuf = user_feats.shape[1]
    nif = item_feats.shape[1]
    bspec = lambda shape: pl.BlockSpec(shape, lambda i: (i, 0))
    wspec = lambda shape: pl.BlockSpec(shape, lambda i: (0, 0))
    out = pl.pallas_call(
        _tc_body,
        grid=(NB,),
        in_specs=[
            bspec((BB, nuf)),
            bspec((BB, nif)),
            bspec((BB, F)),
            bspec((BB, F)),
            bspec((BB, R)),
            bspec((BB, 1)),
            bspec((BB, 1)),
            wspec((nuf, F)),
            wspec((nif, F)),
            wspec((R, F)),
            wspec((1, nuf)),
            wspec((1, nif)),
            wspec((1, 1)),
        ],
        out_specs=bspec((BB, 1)),
        out_shape=jax.ShapeDtypeStruct((B, 1), jnp.float32),
    )(user_feats, item_feats, ue, wiv, a,
      ub.reshape(B, 1), ib.reshape(B, 1),
      W_ufeat, W_ifeat, Bmat,
      user_feat_bias.reshape(1, nuf), item_feat_bias.reshape(1, nif),
      offset.reshape(1, 1))
    return out.reshape(B)
